# baseline (speedup 1.0000x reference)
import functools

import jax
import jax.numpy as jnp
import numpy as np
from jax import lax
from jax.experimental import pallas as pl
from jax.experimental.pallas import tpu as pltpu
from jax.experimental.pallas import tpu_sc as plsc

_NC = 2
_NS = 16
_NW = _NC * _NS
_CH = 128
_BN = 1000
_BE = 8192
_G = 64
_SCRAP = 16

_PERM = np.concatenate(
    [np.concatenate([np.arange(g * 32, (g + 1) * 32, 2),
                     np.arange(g * 32 + 1, (g + 1) * 32, 2)])
     for g in range(4)])


def _prep_kernel(x_ref, pos_ref, wn0, bn0, wn1, bn1, w0x, b0, a_ref, t_ref):
    xb = x_ref[...]
    h = jnp.dot(jax.nn.relu(jnp.dot(xb, wn0[...]) + bn0[...]), wn1[...]) + bn1[...]
    p4 = pos_ref[...]
    nrm = jnp.sqrt(jnp.sum(p4 * p4, axis=1, keepdims=True))
    n4 = p4 / (nrm + 1e-12)
    t_ref[:, 0:4] = p4
    t_ref[:, 4:8] = n4
    t_ref[:, 8:16] = jnp.zeros_like(t_ref[:, 8:16])
    a_ref[...] = (jnp.dot(h, w0x[...]) + b0[...]).astype(jnp.bfloat16)


def _wprep_kernel(wl1_ref, wg0_ref, bl1_ref, m_ref, c_ref):
    wg0 = wg0_ref[0]
    m_ref[0] = jnp.dot(wl1_ref[0], wg0)
    c_ref[0] = jnp.dot(bl1_ref[0], wg0)


def _feat_kernel(ps_ref, pd_ref, w0f0, w0f1, w0f2, c0_ref, c1_ref, c2_ref):
    S = ps_ref[...].T
    D = pd_ref[...].T
    ps, ns = S[0:3, :], S[4:7, :]
    pd, nd = D[0:3, :], D[4:7, :]
    pseudo = ps - pd

    def roll1(v):
        return jnp.concatenate([v[1:3], v[0:1]], axis=0)

    def ang(v1, v2):
        a1, b1 = roll1(v1), roll1(v2)
        a2, b2 = roll1(a1), roll1(b1)
        c = a1 * b2 - a2 * b1
        cn = jnp.sqrt(jnp.sum(c * c, axis=0, keepdims=True))
        dt = jnp.sum(v1 * v2, axis=0, keepdims=True)
        return jnp.arctan2(cn, dt)

    f0 = jnp.sqrt(jnp.sum(pseudo * pseudo, axis=0, keepdims=True))
    ff = jnp.concatenate(
        [f0, ang(nd, pseudo), ang(ns, pseudo), ang(nd, ns)], axis=0).T
    c0_ref[...] = jnp.dot(ff, w0f0[...]).astype(jnp.bfloat16)
    c1_ref[...] = jnp.dot(ff, w0f1[...]).astype(jnp.bfloat16)
    c2_ref[...] = jnp.dot(ff, w0f2[...]).astype(jnp.bfloat16)


def _node_update(s, deg, m, cvec, bg0, wg1, bg1):
    u = jnp.dot(s, m) + deg * cvec + bg0
    return jax.nn.relu(jnp.dot(jax.nn.relu(u), wg1) + bg1)


def _layer0_kernel(sp_ref, degp_ref, m_ref, c_ref, bg0_ref, wg1_ref, bg1_ref,
                   w0x_ref, b0_ref, a_ref, deg8_ref):
    s = sp_ref[0] + sp_ref[1]
    dg8 = degp_ref[0, :, 0:8] + degp_ref[1, :, 0:8]
    h = _node_update(s, dg8[:, 0:1], m_ref[...], c_ref[...],
                     bg0_ref[...], wg1_ref[...], bg1_ref[...])
    a_ref[...] = (jnp.dot(h, w0x_ref[...]) + b0_ref[...]).astype(jnp.bfloat16)
    deg8_ref[...] = dg8


def _layer1_kernel(sp_ref, deg8_ref, m_ref, c_ref, bg0_ref, wg1_ref, bg1_ref,
                   w0x_ref, b0_ref, a_ref):
    s = sp_ref[0] + sp_ref[1]
    h = _node_update(s, deg8_ref[:, 0:1], m_ref[...], c_ref[...],
                     bg0_ref[...], wg1_ref[...], bg1_ref[...])
    a_ref[...] = (jnp.dot(h, w0x_ref[...]) + b0_ref[...]).astype(jnp.bfloat16)


def _layer2_kernel(sp_ref, deg8_ref, m_ref, c_ref, bg0_ref, wg1_ref, bg1_ref,
                   w1r_ref, b1r_ref, w2r_ref, b2r_ref, batch_ref, out_ref):
    s = sp_ref[0] + sp_ref[1]
    h = _node_update(s, deg8_ref[:, 0:1], m_ref[...], c_ref[...],
                     bg0_ref[...], wg1_ref[...], bg1_ref[...])
    z = jax.nn.relu(jnp.dot(h, w1r_ref[...]) + b1r_ref[...])
    p = jnp.dot(z, w2r_ref[...]) + b2r_ref[...]
    oh = (batch_ref[...] == lax.broadcasted_iota(
        jnp.int32, (p.shape[0], _G), 1)).astype(jnp.float32)
    contrib = lax.dot_general(oh, p, (((0,), (0,)), ((), ())))

    @pl.when(pl.program_id(0) == 0)
    def _():
        out_ref[...] = contrib

    @pl.when(pl.program_id(0) != 0)
    def _():
        out_ref[...] += contrib


def _make_pair_gather(n, e):
    mesh = plsc.VectorSubcoreMesh(core_axis_name="c", subcore_axis_name="s")
    nchunk = e // _CH
    npad = n + _SCRAP
    nzchunk = npad // 16
    nochunk = n // 16
    f32 = jnp.float32

    @functools.partial(
        pl.kernel,
        out_type=(jax.ShapeDtypeStruct((e, 16), f32),
                  jax.ShapeDtypeStruct((e, 16), f32),
                  jax.ShapeDtypeStruct((_NC, n, 16), f32)),
        mesh=mesh,
        compiler_params=pltpu.CompilerParams(use_tc_tiling_on_sc=False),
        scratch_types=[
            pltpu.VMEM_SHARED((npad, 16), f32),
            pltpu.VMEM((_CH,), jnp.int32),
            pltpu.VMEM((_CH,), jnp.int32),
            pltpu.VMEM((_CH, 16), f32),
            pltpu.VMEM((_CH, 16), f32),
            pltpu.VMEM((_CH, 16), f32),
            pltpu.SemaphoreType.DMA,
            pltpu.SemaphoreType.DMA,
            pltpu.SemaphoreType.DMA,
        ],
    )
    def k(t_hbm, src_hbm, dst_hbm, ps_hbm, pd_hbm, deg_hbm,
          deg_sh, sidx, didx, srows, drows, ones, sem1, sem2, sem3):
        cid = lax.axis_index("c")
        sid = lax.axis_index("s")
        wid = sid * _NC + cid
        one16 = jnp.ones((16,), f32)
        zero16 = jnp.zeros((16,), f32)

        def orow(r, carry):
            ones[r, pl.ds(0, 16)] = one16
            return carry

        lax.fori_loop(0, _CH, orow, 0)

        def zrow(r, carry):
            srows[r, pl.ds(0, 16)] = zero16
            return carry

        lax.fori_loop(0, 16, zrow, 0)
        zcnt = (nzchunk - sid + _NS - 1) // _NS

        def zbody(ci, carry):
            off = pl.multiple_of((sid + ci * _NS) * 16, 16)
            pltpu.sync_copy(srows.at[pl.ds(0, 16)], deg_sh.at[pl.ds(off, 16)])
            return carry

        lax.fori_loop(0, zcnt, zbody, 0)
        plsc.subcore_barrier()

        nch = nchunk // _NW

        def body(ci, carry):
            base = (wid + ci * _NW) * _CH
            pltpu.sync_copy(src_hbm.at[pl.ds(base, _CH)], sidx)
            pltpu.sync_copy(dst_hbm.at[pl.ds(base, _CH)], didx)
            cp1 = pltpu.async_copy(t_hbm.at[sidx], srows, sem1)
            cp2 = pltpu.async_copy(t_hbm.at[didx], drows, sem2)
            cp3 = pltpu.async_copy(ones, deg_sh.at[didx], sem3, add=True)
            cp1.wait()
            cp2.wait()
            pltpu.sync_copy(srows, ps_hbm.at[pl.ds(base, _CH)])
            pltpu.sync_copy(drows, pd_hbm.at[pl.ds(base, _CH)])
            cp3.wait()
            return carry

        lax.fori_loop(0, nch, body, 0)
        plsc.subcore_barrier()
        ocnt = (nochunk - sid + _NS - 1) // _NS

        def obody(ci, carry):
            off = pl.multiple_of((sid + ci * _NS) * 16, 16)
            pltpu.sync_copy(deg_sh.at[pl.ds(off, 16)],
                            deg_hbm.at[cid, pl.ds(off, 16)])
            return carry

        lax.fori_loop(0, ocnt, obody, 0)

    return k


def _make_edge(n, e):
    mesh = plsc.VectorSubcoreMesh(core_axis_name="c", subcore_axis_name="s")
    nchunk = e // _CH
    nch = nchunk // _NW
    npad = n + _SCRAP
    nzchunk = npad // 16
    nochunk = n // 16
    f32 = jnp.float32
    bf16 = jnp.bfloat16
    assert nch % 2 == 0 and nch >= 4

    @functools.partial(
        pl.kernel,
        out_type=jax.ShapeDtypeStruct((_NC, n, 128), f32),
        mesh=mesh,
        compiler_params=pltpu.CompilerParams(use_tc_tiling_on_sc=False),
        scratch_types=[
            pltpu.VMEM_SHARED((npad, 128), f32),
            pltpu.VMEM((_CH,), jnp.int32),
            pltpu.VMEM((_CH,), jnp.int32),
            pltpu.VMEM((_CH,), jnp.int32),
            pltpu.VMEM((_CH,), jnp.int32),
            pltpu.VMEM((_CH, 64), jnp.int32),
            pltpu.VMEM((_CH, 64), jnp.int32),
            pltpu.VMEM((_CH, 64), jnp.int32),
            pltpu.VMEM((_CH, 64), jnp.int32),
            pltpu.VMEM((_CH, 128), f32),
            pltpu.SemaphoreType.DMA,
            pltpu.SemaphoreType.DMA,
            pltpu.SemaphoreType.DMA,
            pltpu.SemaphoreType.DMA,
            pltpu.SemaphoreType.DMA,
            pltpu.SemaphoreType.DMA,
            pltpu.SemaphoreType.DMA,
        ],
    )
    def k(a_hbm, src_hbm, dst_hbm, c_hbm, out_hbm, s_sh,
          srcv0, srcv1, dstv0, dstv1, agath0, agath1, cbuf0, cbuf1, msg,
          gsem0, gsem1, csem0, csem1, dsem0, dsem1, ssem):
        SR, DV = [srcv0, srcv1], [dstv0, dstv1]
        AG, CB = [agath0, agath1], [cbuf0, cbuf1]
        GS, CS, DS = [gsem0, gsem1], [csem0, csem1], [dsem0, dsem1]
        cid = lax.axis_index("c")
        sid = lax.axis_index("s")
        wid = sid * _NC + cid
        zero16 = jnp.zeros((16,), f32)

        def cbase(ci):
            return (wid + ci * _NW) * _CH

        def fetch_main(ci, p):
            b = cbase(ci)
            pltpu.sync_copy(src_hbm.at[pl.ds(b, _CH)], SR[p])
            pltpu.async_copy(a_hbm.at[SR[p]], AG[p], GS[p])
            pltpu.async_copy(c_hbm.at[pl.ds(b, _CH)], CB[p], CS[p])

        def fetch_dst(ci, p):
            pltpu.async_copy(dst_hbm.at[pl.ds(cbase(ci), _CH)], DV[p], DS[p])

        def wait_fetch(p):
            pltpu.make_async_copy(a_hbm.at[SR[p]], AG[p], GS[p]).wait()
            pltpu.make_async_copy(c_hbm.at[pl.ds(0, _CH)], CB[p], CS[p]).wait()

        def wait_dst(p):
            pltpu.make_async_copy(dst_hbm.at[pl.ds(0, _CH)], DV[p], DS[p]).wait()

        def issue_scatter(p):
            pltpu.async_copy(msg, s_sh.at[DV[p]], ssem, add=True)

        def wait_scatter(p):
            pltpu.make_async_copy(msg, s_sh.at[DV[p]], ssem).wait()

        def zrow(r, carry):
            for j in range(8):
                msg[r, pl.ds(j * 16, 16)] = zero16
            return carry

        lax.fori_loop(0, 16, zrow, 0)
        zcnt = (nzchunk - sid + _NS - 1) // _NS

        def zbody(ci, carry):
            off = pl.multiple_of((sid + ci * _NS) * 16, 16)
            pltpu.sync_copy(msg.at[pl.ds(0, 16)], s_sh.at[pl.ds(off, 16)])
            return carry

        lax.fori_loop(0, zcnt, zbody, 0)
        plsc.subcore_barrier()

        himask = jnp.full((16,), -65536, jnp.int32)

        def compute(p):
            agath, cbuf = AG[p], CB[p]

            def ebody(i2, carry):
                for u in range(2):
                    i = i2 * 2 + u
                    for g in range(4):
                        a = agath[i, pl.ds(g * 16, 16)]
                        c = cbuf[i, pl.ds(g * 16, 16)]
                        bc = lambda v: lax.bitcast_convert_type(v, jnp.float32)
                        alo = bc(a << 16)
                        ahi = bc(a & himask)
                        clo = bc(c << 16)
                        chi = bc(c & himask)
                        msg[i, pl.ds(g * 32, 16)] = jnp.maximum(
                            alo + clo, 0.0)
                        msg[i, pl.ds(g * 32 + 16, 16)] = jnp.maximum(
                            ahi + chi, 0.0)
                return carry

            lax.fori_loop(0, _CH // 2, ebody, 0)

        fetch_main(0, 0)
        fetch_dst(0, 0)

        def pipe(i, carry):
            for p in (0, 1):
                ci = 2 * i + p
                nxt = 1 - p

                @pl.when(ci + 1 < nch)
                def _():
                    fetch_main(ci + 1, nxt)
                wait_fetch(p)

                @pl.when(ci > 0)
                def _():
                    wait_scatter(nxt)

                @pl.when(ci + 1 < nch)
                def _():
                    fetch_dst(ci + 1, nxt)
                compute(p)
                wait_dst(p)
                issue_scatter(p)
            return carry

        lax.fori_loop(0, nch // 2, pipe, 0)
        wait_scatter(1)
        plsc.subcore_barrier()

        ocnt = (nochunk - sid + _NS - 1) // _NS

        def obody(ci, carry):
            off = pl.multiple_of((sid + ci * _NS) * 16, 16)
            pltpu.sync_copy(s_sh.at[pl.ds(off, 16)],
                            out_hbm.at[cid, pl.ds(off, 16)])
            return carry

        lax.fori_loop(0, ocnt, obody, 0)

    return k


def _row(v):
    return v.reshape(1, -1)


def kernel(x, pos, edge_index, batch, params):
    n = x.shape[0]
    e = edge_index.shape[1]
    nb = n // _BN
    f32 = jnp.float32
    estep = _CH * _NW
    epad = ((e + estep - 1) // estep) * estep
    nch_w = epad // estep
    if nch_w % 4 != 0:
        epad += (4 - nch_w % 4) * estep
    src = edge_index[0].astype(jnp.int32)
    dst = edge_index[1].astype(jnp.int32)
    npadv = epad - e
    src_p = jnp.concatenate([src, jnp.zeros((npadv,), jnp.int32)])
    dst_p = jnp.concatenate(
        [dst, n + (jnp.arange(npadv, dtype=jnp.int32) % _SCRAP)])
    pos4 = jnp.pad(pos.astype(f32), ((0, 0), (0, 1)))

    wn0, bn0, wn1, bn1 = params["node_lin"]
    loc = params["local"]
    glo = params["global"]
    w0x = [loc[i][0][:128] for i in range(3)]
    w0f = [loc[i][0][128:] for i in range(3)]
    b0l = [loc[i][1] for i in range(3)]
    w1r, b1r = params["lin1"]
    w2r, b2r = params["lin2"]

    full = lambda shp: pl.BlockSpec(shp, lambda i: tuple(0 for _ in shp))
    nblk = lambda shp: pl.BlockSpec(shp, lambda i: (i,) + tuple(0 for _ in shp[1:]))

    a0, tbl = pl.pallas_call(
        _prep_kernel,
        grid=(nb,),
        in_specs=[nblk((_BN, 128)), nblk((_BN, 4)), full((128, 128)),
                  full((1, 128)), full((128, 128)), full((1, 128)),
                  full((128, 128)), full((1, 128))],
        out_specs=[nblk((_BN, 128)), nblk((_BN, 16))],
        out_shape=[jax.ShapeDtypeStruct((n, 128), jnp.bfloat16),
                   jax.ShapeDtypeStruct((n, 16), f32)],
    )(x, pos4, wn0, _row(bn0), wn1, _row(bn1), w0x[0], _row(b0l[0]))

    wl1s = jnp.stack([loc[i][2][_PERM, :] for i in range(3)])
    wg0s = jnp.stack([glo[i][0] for i in range(3)])
    bl1s = jnp.stack([_row(loc[i][3]) for i in range(3)])
    ms, cs = pl.pallas_call(
        _wprep_kernel,
        grid=(3,),
        in_specs=[pl.BlockSpec((1, 128, 128), lambda i: (i, 0, 0)),
                  pl.BlockSpec((1, 128, 128), lambda i: (i, 0, 0)),
                  pl.BlockSpec((1, 1, 128), lambda i: (i, 0, 0))],
        out_specs=[pl.BlockSpec((1, 128, 128), lambda i: (i, 0, 0)),
                   pl.BlockSpec((1, 1, 128), lambda i: (i, 0, 0))],
        out_shape=[jax.ShapeDtypeStruct((3, 128, 128), f32),
                   jax.ShapeDtypeStruct((3, 1, 128), f32)],
    )(wl1s, wg0s, bl1s)

    prs, prd, degp = _make_pair_gather(n, epad)(tbl, src_p, dst_p)

    bf16 = jnp.bfloat16
    cc0, cc1, cc2 = pl.pallas_call(
        _feat_kernel,
        grid=(epad // _BE,),
        in_specs=[nblk((_BE, 16)), nblk((_BE, 16)),
                  full((4, 128)), full((4, 128)), full((4, 128))],
        out_specs=[nblk((_BE, 128))] * 3,
        out_shape=[jax.ShapeDtypeStruct((epad, 128), bf16)] * 3,
    )(prs, prd, w0f[0], w0f[1], w0f[2])

    edge = _make_edge(n, epad)
    r3 = lambda t: lax.bitcast_convert_type(
        t.reshape(t.shape[0], 64, 2), jnp.int32)

    sp0 = edge(r3(a0), src_p, dst_p, r3(cc0))
    a1, deg8 = pl.pallas_call(
        _layer0_kernel,
        grid=(nb,),
        in_specs=[pl.BlockSpec((2, _BN, 128), lambda i: (0, i, 0)),
                  pl.BlockSpec((2, _BN, 16), lambda i: (0, i, 0)),
                  full((128, 128)), full((1, 128)), full((1, 128)),
                  full((128, 128)), full((1, 128)),
                  full((128, 128)), full((1, 128))],
        out_specs=[nblk((_BN, 128)), nblk((_BN, 8))],
        out_shape=[jax.ShapeDtypeStruct((n, 128), bf16),
                   jax.ShapeDtypeStruct((n, 8), f32)],
    )(sp0, degp, ms[0], cs[0], _row(glo[0][1]), glo[0][2], _row(glo[0][3]),
      w0x[1], _row(b0l[1]))

    sp1 = edge(r3(a1), src_p, dst_p, r3(cc1))
    a2 = pl.pallas_call(
        _layer1_kernel,
        grid=(nb,),
        in_specs=[pl.BlockSpec((2, _BN, 128), lambda i: (0, i, 0)),
                  nblk((_BN, 8)),
                  full((128, 128)), full((1, 128)), full((1, 128)),
                  full((128, 128)), full((1, 128)),
                  full((128, 128)), full((1, 128))],
        out_specs=nblk((_BN, 128)),
        out_shape=jax.ShapeDtypeStruct((n, 128), bf16),
    )(sp1, deg8, ms[1], cs[1], _row(glo[1][1]), glo[1][2], _row(glo[1][3]),
      w0x[2], _row(b0l[2]))

    sp2 = edge(r3(a2), src_p, dst_p, r3(cc2))
    out = pl.pallas_call(
        _layer2_kernel,
        grid=(nb,),
        in_specs=[pl.BlockSpec((2, _BN, 128), lambda i: (0, i, 0)),
                  nblk((_BN, 8)),
                  full((128, 128)), full((1, 128)), full((1, 128)),
                  full((128, 128)), full((1, 128)),
                  full((128, 64)), full((1, 64)), full((64, 128)),
                  full((1, 128)), nblk((_BN, 1))],
        out_specs=pl.BlockSpec((_G, 128), lambda i: (0, 0)),
        out_shape=jax.ShapeDtypeStruct((_G, 128), f32),
    )(sp2, deg8, ms[2], cs[2], _row(glo[2][1]), glo[2][2], _row(glo[2][3]),
      w1r, _row(b1r), w2r, _row(b2r),
      batch.astype(jnp.int32).reshape(n, 1))
    return out


# speedup vs baseline: 1.8898x; 1.8898x over previous
import functools

import jax
import jax.numpy as jnp
import numpy as np
from jax import lax
from jax.experimental import pallas as pl
from jax.experimental.pallas import tpu as pltpu
from jax.experimental.pallas import tpu_sc as plsc

_NC = 2
_NS = 16
_NW = _NC * _NS
_CH = 128
_BN = 1000
_BE = 8192
_G = 64
_SCRAP = 16

_PERM = np.concatenate(
    [np.concatenate([np.arange(16 * g, 16 * g + 16),
                     np.arange(64 + 16 * g, 64 + 16 * g + 16)])
     for g in range(4)])


def _pack_words(x):
    w = lax.bitcast_convert_type(
        x.astype(jnp.bfloat16), jnp.int16).astype(jnp.int32)
    return (w[:, :64] & 0xFFFF) | (w[:, 64:] << 16)


def _prep_kernel(x_ref, pos_ref, wn0, bn0, wn1, bn1, w0x, b0, a_ref, t_ref):
    xb = x_ref[...]
    h = jnp.dot(jax.nn.relu(jnp.dot(xb, wn0[...]) + bn0[...]), wn1[...]) + bn1[...]
    p4 = pos_ref[...]
    nrm = jnp.sqrt(jnp.sum(p4 * p4, axis=1, keepdims=True))
    n4 = p4 / (nrm + 1e-12)
    t_ref[:, 0:4] = p4
    t_ref[:, 4:8] = n4
    t_ref[:, 8:16] = jnp.zeros_like(t_ref[:, 8:16])
    a_ref[...] = _pack_words(jnp.dot(h, w0x[...]) + b0[...])


def _wprep_kernel(wl1_ref, wg0_ref, bl1_ref, m_ref, c_ref):
    wg0 = wg0_ref[0]
    m_ref[0] = jnp.dot(wl1_ref[0], wg0)
    c_ref[0] = jnp.dot(bl1_ref[0], wg0)


def _feat_kernel(ps_ref, pd_ref, w0f0, w0f1, w0f2, c0_ref, c1_ref, c2_ref):
    S = ps_ref[...].T
    D = pd_ref[...].T
    ps, ns = S[0:3, :], S[4:7, :]
    pd, nd = D[0:3, :], D[4:7, :]
    pseudo = ps - pd

    def roll1(v):
        return jnp.concatenate([v[1:3], v[0:1]], axis=0)

    def ang(v1, v2):
        a1, b1 = roll1(v1), roll1(v2)
        a2, b2 = roll1(a1), roll1(b1)
        c = a1 * b2 - a2 * b1
        cn = jnp.sqrt(jnp.sum(c * c, axis=0, keepdims=True))
        dt = jnp.sum(v1 * v2, axis=0, keepdims=True)
        return jnp.arctan2(cn, dt)

    f0 = jnp.sqrt(jnp.sum(pseudo * pseudo, axis=0, keepdims=True))
    ff = jnp.concatenate(
        [f0, ang(nd, pseudo), ang(ns, pseudo), ang(nd, ns)], axis=0).T
    c0_ref[...] = _pack_words(jnp.dot(ff, w0f0[...]))
    c1_ref[...] = _pack_words(jnp.dot(ff, w0f1[...]))
    c2_ref[...] = _pack_words(jnp.dot(ff, w0f2[...]))


def _node_update(s, deg, m, cvec, bg0, wg1, bg1):
    u = jnp.dot(s, m) + deg * cvec + bg0
    return jax.nn.relu(jnp.dot(jax.nn.relu(u), wg1) + bg1)


def _layer0_kernel(sp_ref, degp_ref, m_ref, c_ref, bg0_ref, wg1_ref, bg1_ref,
                   w0x_ref, b0_ref, a_ref, deg8_ref):
    s = sp_ref[0] + sp_ref[1]
    dg8 = degp_ref[0, :, 0:8] + degp_ref[1, :, 0:8]
    h = _node_update(s, dg8[:, 0:1], m_ref[...], c_ref[...],
                     bg0_ref[...], wg1_ref[...], bg1_ref[...])
    a_ref[...] = _pack_words(jnp.dot(h, w0x_ref[...]) + b0_ref[...])
    deg8_ref[...] = dg8


def _layer1_kernel(sp_ref, deg8_ref, m_ref, c_ref, bg0_ref, wg1_ref, bg1_ref,
                   w0x_ref, b0_ref, a_ref):
    s = sp_ref[0] + sp_ref[1]
    h = _node_update(s, deg8_ref[:, 0:1], m_ref[...], c_ref[...],
                     bg0_ref[...], wg1_ref[...], bg1_ref[...])
    a_ref[...] = _pack_words(jnp.dot(h, w0x_ref[...]) + b0_ref[...])


def _layer2_kernel(sp_ref, deg8_ref, m_ref, c_ref, bg0_ref, wg1_ref, bg1_ref,
                   w1r_ref, b1r_ref, w2r_ref, b2r_ref, batch_ref, out_ref):
    s = sp_ref[0] + sp_ref[1]
    h = _node_update(s, deg8_ref[:, 0:1], m_ref[...], c_ref[...],
                     bg0_ref[...], wg1_ref[...], bg1_ref[...])
    z = jax.nn.relu(jnp.dot(h, w1r_ref[...]) + b1r_ref[...])
    p = jnp.dot(z, w2r_ref[...]) + b2r_ref[...]
    oh = (batch_ref[...] == lax.broadcasted_iota(
        jnp.int32, (p.shape[0], _G), 1)).astype(jnp.float32)
    contrib = lax.dot_general(oh, p, (((0,), (0,)), ((), ())))

    @pl.when(pl.program_id(0) == 0)
    def _():
        out_ref[...] = contrib

    @pl.when(pl.program_id(0) != 0)
    def _():
        out_ref[...] += contrib


def _make_pair_gather(n, e):
    mesh = plsc.VectorSubcoreMesh(core_axis_name="c", subcore_axis_name="s")
    nchunk = e // _CH
    npad = n + _SCRAP
    nzchunk = npad // 16
    nochunk = n // 16
    f32 = jnp.float32

    @functools.partial(
        pl.kernel,
        out_type=(jax.ShapeDtypeStruct((e, 16), f32),
                  jax.ShapeDtypeStruct((e, 16), f32),
                  jax.ShapeDtypeStruct((_NC, n, 16), f32)),
        mesh=mesh,
        compiler_params=pltpu.CompilerParams(use_tc_tiling_on_sc=False),
        scratch_types=[
            pltpu.VMEM_SHARED((npad, 16), f32),
            pltpu.VMEM((_CH,), jnp.int32),
            pltpu.VMEM((_CH,), jnp.int32),
            pltpu.VMEM((_CH, 16), f32),
            pltpu.VMEM((_CH, 16), f32),
            pltpu.VMEM((_CH, 16), f32),
            pltpu.SemaphoreType.DMA,
            pltpu.SemaphoreType.DMA,
            pltpu.SemaphoreType.DMA,
        ],
    )
    def k(t_hbm, src_hbm, dst_hbm, ps_hbm, pd_hbm, deg_hbm,
          deg_sh, sidx, didx, srows, drows, ones, sem1, sem2, sem3):
        cid = lax.axis_index("c")
        sid = lax.axis_index("s")
        wid = sid * _NC + cid
        one16 = jnp.ones((16,), f32)
        zero16 = jnp.zeros((16,), f32)

        def orow(r, carry):
            ones[r, pl.ds(0, 16)] = one16
            return carry

        lax.fori_loop(0, _CH, orow, 0)

        def zrow(r, carry):
            srows[r, pl.ds(0, 16)] = zero16
            return carry

        lax.fori_loop(0, 16, zrow, 0)
        zcnt = (nzchunk - sid + _NS - 1) // _NS

        def zbody(ci, carry):
            off = pl.multiple_of((sid + ci * _NS) * 16, 16)
            pltpu.sync_copy(srows.at[pl.ds(0, 16)], deg_sh.at[pl.ds(off, 16)])
            return carry

        lax.fori_loop(0, zcnt, zbody, 0)
        plsc.subcore_barrier()

        nch = nchunk // _NW

        def body(ci, carry):
            base = (wid + ci * _NW) * _CH
            pltpu.sync_copy(src_hbm.at[pl.ds(base, _CH)], sidx)
            pltpu.sync_copy(dst_hbm.at[pl.ds(base, _CH)], didx)
            cp1 = pltpu.async_copy(t_hbm.at[sidx], srows, sem1)
            cp2 = pltpu.async_copy(t_hbm.at[didx], drows, sem2)
            cp3 = pltpu.async_copy(ones, deg_sh.at[didx], sem3, add=True)
            cp1.wait()
            cp2.wait()
            pltpu.sync_copy(srows, ps_hbm.at[pl.ds(base, _CH)])
            pltpu.sync_copy(drows, pd_hbm.at[pl.ds(base, _CH)])
            cp3.wait()
            return carry

        lax.fori_loop(0, nch, body, 0)
        plsc.subcore_barrier()
        ocnt = (nochunk - sid + _NS - 1) // _NS

        def obody(ci, carry):
            off = pl.multiple_of((sid + ci * _NS) * 16, 16)
            pltpu.sync_copy(deg_sh.at[pl.ds(off, 16)],
                            deg_hbm.at[cid, pl.ds(off, 16)])
            return carry

        lax.fori_loop(0, ocnt, obody, 0)

    return k


def _make_edge(n, e):
    mesh = plsc.VectorSubcoreMesh(core_axis_name="c", subcore_axis_name="s")
    nchunk = e // _CH
    nch = nchunk // _NW
    npad = n + _SCRAP
    nzchunk = npad // 16
    nochunk = n // 16
    f32 = jnp.float32
    bf16 = jnp.bfloat16
    assert nch % 2 == 0 and nch >= 4

    @functools.partial(
        pl.kernel,
        out_type=jax.ShapeDtypeStruct((_NC, n, 128), f32),
        mesh=mesh,
        compiler_params=pltpu.CompilerParams(use_tc_tiling_on_sc=False),
        scratch_types=[
            pltpu.VMEM_SHARED((npad, 128), f32),
            pltpu.VMEM((_CH,), jnp.int32),
            pltpu.VMEM((_CH,), jnp.int32),
            pltpu.VMEM((_CH,), jnp.int32),
            pltpu.VMEM((_CH,), jnp.int32),
            pltpu.VMEM((_CH, 64), jnp.int32),
            pltpu.VMEM((_CH, 64), jnp.int32),
            pltpu.VMEM((_CH, 64), jnp.int32),
            pltpu.VMEM((_CH, 64), jnp.int32),
            pltpu.VMEM((_CH, 128), f32),
            pltpu.SemaphoreType.DMA,
            pltpu.SemaphoreType.DMA,
            pltpu.SemaphoreType.DMA,
            pltpu.SemaphoreType.DMA,
            pltpu.SemaphoreType.DMA,
            pltpu.SemaphoreType.DMA,
            pltpu.SemaphoreType.DMA,
        ],
    )
    def k(a_hbm, src_hbm, dst_hbm, c_hbm, out_hbm, s_sh,
          srcv0, srcv1, dstv0, dstv1, agath0, agath1, cbuf0, cbuf1, msg,
          gsem0, gsem1, csem0, csem1, dsem0, dsem1, ssem):
        SR, DV = [srcv0, srcv1], [dstv0, dstv1]
        AG, CB = [agath0, agath1], [cbuf0, cbuf1]
        GS, CS, DS = [gsem0, gsem1], [csem0, csem1], [dsem0, dsem1]
        cid = lax.axis_index("c")
        sid = lax.axis_index("s")
        wid = sid * _NC + cid
        zero16 = jnp.zeros((16,), f32)

        def cbase(ci):
            return (wid + ci * _NW) * _CH

        def fetch_main(ci, p):
            b = cbase(ci)
            pltpu.sync_copy(src_hbm.at[pl.ds(b, _CH)], SR[p])
            pltpu.async_copy(a_hbm.at[SR[p]], AG[p], GS[p])
            pltpu.async_copy(c_hbm.at[pl.ds(b, _CH)], CB[p], CS[p])

        def fetch_dst(ci, p):
            pltpu.async_copy(dst_hbm.at[pl.ds(cbase(ci), _CH)], DV[p], DS[p])

        def wait_fetch(p):
            pltpu.make_async_copy(a_hbm.at[SR[p]], AG[p], GS[p]).wait()
            pltpu.make_async_copy(c_hbm.at[pl.ds(0, _CH)], CB[p], CS[p]).wait()

        def wait_dst(p):
            pltpu.make_async_copy(dst_hbm.at[pl.ds(0, _CH)], DV[p], DS[p]).wait()

        def issue_scatter(p):
            pltpu.async_copy(msg, s_sh.at[DV[p]], ssem, add=True)

        def wait_scatter(p):
            pltpu.make_async_copy(msg, s_sh.at[DV[p]], ssem).wait()

        def zrow(r, carry):
            for j in range(8):
                msg[r, pl.ds(j * 16, 16)] = zero16
            return carry

        lax.fori_loop(0, 16, zrow, 0)
        zcnt = (nzchunk - sid + _NS - 1) // _NS

        def zbody(ci, carry):
            off = pl.multiple_of((sid + ci * _NS) * 16, 16)
            pltpu.sync_copy(msg.at[pl.ds(0, 16)], s_sh.at[pl.ds(off, 16)])
            return carry

        lax.fori_loop(0, zcnt, zbody, 0)
        plsc.subcore_barrier()

        himask = jnp.full((16,), -65536, jnp.int32)

        def compute(p):
            agath, cbuf = AG[p], CB[p]

            def ebody(i2, carry):
                for u in range(2):
                    i = i2 * 2 + u
                    for g in range(4):
                        a = agath[i, pl.ds(g * 16, 16)]
                        c = cbuf[i, pl.ds(g * 16, 16)]
                        bc = lambda v: lax.bitcast_convert_type(v, jnp.float32)
                        alo = bc(a << 16)
                        ahi = bc(a & himask)
                        clo = bc(c << 16)
                        chi = bc(c & himask)
                        msg[i, pl.ds(g * 32, 16)] = jnp.maximum(
                            alo + clo, 0.0)
                        msg[i, pl.ds(g * 32 + 16, 16)] = jnp.maximum(
                            ahi + chi, 0.0)
                return carry

            lax.fori_loop(0, _CH // 2, ebody, 0)

        fetch_main(0, 0)
        fetch_dst(0, 0)

        def pipe(i, carry):
            for p in (0, 1):
                ci = 2 * i + p
                nxt = 1 - p

                @pl.when(ci + 1 < nch)
                def _():
                    fetch_main(ci + 1, nxt)
                wait_fetch(p)

                @pl.when(ci > 0)
                def _():
                    wait_scatter(nxt)

                @pl.when(ci + 1 < nch)
                def _():
                    fetch_dst(ci + 1, nxt)
                compute(p)
                wait_dst(p)
                issue_scatter(p)
            return carry

        lax.fori_loop(0, nch // 2, pipe, 0)
        wait_scatter(1)
        plsc.subcore_barrier()

        ocnt = (nochunk - sid + _NS - 1) // _NS

        def obody(ci, carry):
            off = pl.multiple_of((sid + ci * _NS) * 16, 16)
            pltpu.sync_copy(s_sh.at[pl.ds(off, 16)],
                            out_hbm.at[cid, pl.ds(off, 16)])
            return carry

        lax.fori_loop(0, ocnt, obody, 0)

    return k


def _row(v):
    return v.reshape(1, -1)


def kernel(x, pos, edge_index, batch, params):
    n = x.shape[0]
    e = edge_index.shape[1]
    nb = n // _BN
    f32 = jnp.float32
    estep = _CH * _NW
    epad = ((e + estep - 1) // estep) * estep
    nch_w = epad // estep
    if nch_w % 4 != 0:
        epad += (4 - nch_w % 4) * estep
    src = edge_index[0].astype(jnp.int32)
    dst = edge_index[1].astype(jnp.int32)
    npadv = epad - e
    src_p = jnp.concatenate([src, jnp.zeros((npadv,), jnp.int32)])
    dst_p = jnp.concatenate(
        [dst, n + (jnp.arange(npadv, dtype=jnp.int32) % _SCRAP)])
    pos4 = jnp.pad(pos.astype(f32), ((0, 0), (0, 1)))

    wn0, bn0, wn1, bn1 = params["node_lin"]
    loc = params["local"]
    glo = params["global"]
    w0x = [loc[i][0][:128] for i in range(3)]
    w0f = [loc[i][0][128:] for i in range(3)]
    b0l = [loc[i][1] for i in range(3)]
    w1r, b1r = params["lin1"]
    w2r, b2r = params["lin2"]

    full = lambda shp: pl.BlockSpec(shp, lambda i: tuple(0 for _ in shp))
    nblk = lambda shp: pl.BlockSpec(shp, lambda i: (i,) + tuple(0 for _ in shp[1:]))

    a0, tbl = pl.pallas_call(
        _prep_kernel,
        grid=(nb,),
        in_specs=[nblk((_BN, 128)), nblk((_BN, 4)), full((128, 128)),
                  full((1, 128)), full((128, 128)), full((1, 128)),
                  full((128, 128)), full((1, 128))],
        out_specs=[nblk((_BN, 64)), nblk((_BN, 16))],
        out_shape=[jax.ShapeDtypeStruct((n, 64), jnp.int32),
                   jax.ShapeDtypeStruct((n, 16), f32)],
    )(x, pos4, wn0, _row(bn0), wn1, _row(bn1), w0x[0], _row(b0l[0]))

    wl1s = jnp.stack([loc[i][2][_PERM, :] for i in range(3)])
    wg0s = jnp.stack([glo[i][0] for i in range(3)])
    bl1s = jnp.stack([_row(loc[i][3]) for i in range(3)])
    ms, cs = pl.pallas_call(
        _wprep_kernel,
        grid=(3,),
        in_specs=[pl.BlockSpec((1, 128, 128), lambda i: (i, 0, 0)),
                  pl.BlockSpec((1, 128, 128), lambda i: (i, 0, 0)),
                  pl.BlockSpec((1, 1, 128), lambda i: (i, 0, 0))],
        out_specs=[pl.BlockSpec((1, 128, 128), lambda i: (i, 0, 0)),
                   pl.BlockSpec((1, 1, 128), lambda i: (i, 0, 0))],
        out_shape=[jax.ShapeDtypeStruct((3, 128, 128), f32),
                   jax.ShapeDtypeStruct((3, 1, 128), f32)],
    )(wl1s, wg0s, bl1s)

    prs, prd, degp = _make_pair_gather(n, epad)(tbl, src_p, dst_p)

    bf16 = jnp.bfloat16
    cc0, cc1, cc2 = pl.pallas_call(
        _feat_kernel,
        grid=(epad // _BE,),
        in_specs=[nblk((_BE, 16)), nblk((_BE, 16)),
                  full((4, 128)), full((4, 128)), full((4, 128))],
        out_specs=[nblk((_BE, 64))] * 3,
        out_shape=[jax.ShapeDtypeStruct((epad, 64), jnp.int32)] * 3,
    )(prs, prd, w0f[0], w0f[1], w0f[2])

    edge = _make_edge(n, epad)

    sp0 = edge(a0, src_p, dst_p, cc0)
    a1, deg8 = pl.pallas_call(
        _layer0_kernel,
        grid=(nb,),
        in_specs=[pl.BlockSpec((2, _BN, 128), lambda i: (0, i, 0)),
                  pl.BlockSpec((2, _BN, 16), lambda i: (0, i, 0)),
                  full((128, 128)), full((1, 128)), full((1, 128)),
                  full((128, 128)), full((1, 128)),
                  full((128, 128)), full((1, 128))],
        out_specs=[nblk((_BN, 64)), nblk((_BN, 8))],
        out_shape=[jax.ShapeDtypeStruct((n, 64), jnp.int32),
                   jax.ShapeDtypeStruct((n, 8), f32)],
    )(sp0, degp, ms[0], cs[0], _row(glo[0][1]), glo[0][2], _row(glo[0][3]),
      w0x[1], _row(b0l[1]))

    sp1 = edge(a1, src_p, dst_p, cc1)
    a2 = pl.pallas_call(
        _layer1_kernel,
        grid=(nb,),
        in_specs=[pl.BlockSpec((2, _BN, 128), lambda i: (0, i, 0)),
                  nblk((_BN, 8)),
                  full((128, 128)), full((1, 128)), full((1, 128)),
                  full((128, 128)), full((1, 128)),
                  full((128, 128)), full((1, 128))],
        out_specs=nblk((_BN, 64)),
        out_shape=jax.ShapeDtypeStruct((n, 64), jnp.int32),
    )(sp1, deg8, ms[1], cs[1], _row(glo[1][1]), glo[1][2], _row(glo[1][3]),
      w0x[2], _row(b0l[2]))

    sp2 = edge(a2, src_p, dst_p, cc2)
    out = pl.pallas_call(
        _layer2_kernel,
        grid=(nb,),
        in_specs=[pl.BlockSpec((2, _BN, 128), lambda i: (0, i, 0)),
                  nblk((_BN, 8)),
                  full((128, 128)), full((1, 128)), full((1, 128)),
                  full((128, 128)), full((1, 128)),
                  full((128, 64)), full((1, 64)), full((64, 128)),
                  full((1, 128)), nblk((_BN, 1))],
        out_specs=pl.BlockSpec((_G, 128), lambda i: (0, 0)),
        out_shape=jax.ShapeDtypeStruct((_G, 128), f32),
    )(sp2, deg8, ms[2], cs[2], _row(glo[2][1]), glo[2][2], _row(glo[2][3]),
      w1r, _row(b1r), w2r, _row(b2r),
      batch.astype(jnp.int32).reshape(n, 1))
    return out
